# Initial kernel scaffold; baseline (speedup 1.0000x reference)
import jax
import jax.numpy as jnp
from jax.experimental import pallas as pl


def kernel(X, F, N, Q, W):
    raise NotImplementedError("write your pallas kernel here")


# speedup vs baseline: 3.8354x; 3.8354x over previous
import functools

import jax
import jax.numpy as jnp
from jax import lax
from jax.experimental import pallas as pl
from jax.experimental.pallas import tpu as pltpu
from jax.experimental.pallas import tpu_sc as plsc

_SIGMA = 1.0
_TW = 144
_CHUNK = 128
_NWORKERS = 32
_BPTS = 400
_KAPPA = 32


def _sc_gather(table, idxs):
    etot = idxs.shape[0]
    epw = etot // _NWORKERS
    nchunks = epw // _CHUNK
    mesh = plsc.VectorSubcoreMesh(core_axis_name="c", subcore_axis_name="s")

    @functools.partial(
        pl.kernel,
        mesh=mesh,
        out_type=jax.ShapeDtypeStruct((etot, _TW), jnp.float32),
        scratch_types=[
            pltpu.VMEM((_CHUNK,), jnp.int32),
            pltpu.VMEM((_CHUNK, _TW), jnp.float32),
            pltpu.SemaphoreType.DMA,
        ],
        compiler_params=pltpu.CompilerParams(use_tc_tiling_on_sc=False),
    )
    def gather_kernel(tab_hbm, idx_hbm, out_hbm, idx_v, rows_v, sem):
        wid = lax.axis_index("s") * 2 + lax.axis_index("c")
        base = wid * epw

        def body(j, carry):
            off = base + j * _CHUNK
            pltpu.sync_copy(idx_hbm.at[pl.ds(off, _CHUNK)], idx_v)
            pltpu.async_copy(tab_hbm.at[idx_v], rows_v, sem).wait()
            pltpu.sync_copy(rows_v, out_hbm.at[pl.ds(off, _CHUNK)])
            return carry

        lax.fori_loop(0, nchunks, body, 0)

    return gather_kernel(table, idxs)


def _tc_body(g_ref, x_ref, qt_ref, q2_ref, wf_ref, y_ref):
    b = y_ref.shape[0]
    kappa = g_ref.shape[0] // b
    din = wf_ref.shape[0] // qt_ref.shape[1]
    nx = qt_ref.shape[0]
    mq = qt_ref.shape[1]
    e = b * kappa

    g = g_ref[...]
    fn = g[:, :din]
    xn = g[:, din:din + nx]
    xc = jnp.repeat(x_ref[...], kappa, axis=0)
    diff = xn - xc
    dd = jnp.sum(diff * diff, axis=1, keepdims=True)
    dq = jnp.dot(diff, qt_ref[...],
                 precision=lax.Precision.HIGHEST,
                 preferred_element_type=jnp.float32)
    d2 = jnp.maximum(dd - 2.0 * dq + q2_ref[...], 0.0)
    dist = jnp.sqrt(d2 + 1e-12)
    infl = jnp.maximum(0.0, 1.0 - dist / _SIGMA)

    parts = []
    for m in range(mq):
        w = infl[:, m:m + 1]
        t = (w * fn).reshape(b, kappa, din)
        parts.append(jnp.sum(t, axis=1))
    agg = jnp.concatenate(parts, axis=1)
    y_ref[...] = jnp.dot(agg, wf_ref[...],
                         precision=lax.Precision.HIGHEST,
                         preferred_element_type=jnp.float32)


def _tc_compute(g, x2, qt, q2, wf, r, dout):
    nblocks = r // _BPTS
    eblk = _BPTS * _KAPPA

    return pl.pallas_call(
        _tc_body,
        grid=(nblocks,),
        in_specs=[
            pl.BlockSpec((eblk, _TW), lambda i: (i, 0)),
            pl.BlockSpec((_BPTS, x2.shape[1]), lambda i: (i, 0)),
            pl.BlockSpec(qt.shape, lambda i: (0, 0)),
            pl.BlockSpec(q2.shape, lambda i: (0, 0)),
            pl.BlockSpec(wf.shape, lambda i: (0, 0)),
        ],
        out_specs=pl.BlockSpec((_BPTS, dout), lambda i: (i, 0)),
        out_shape=jax.ShapeDtypeStruct((r, dout), jnp.float32),
        compiler_params=pltpu.CompilerParams(
            dimension_semantics=("arbitrary",),
        ),
    )(g, x2, qt, q2, wf)


def kernel(X, F, N, Q, W):
    k, r, nx = X.shape
    kappa = N.shape[2]
    mq, din, dout = W.shape
    x2 = X[0]
    f2 = F[0]

    pad = jnp.zeros((r, _TW - din - nx), jnp.float32)
    table = jnp.concatenate([f2, x2, pad], axis=1)

    nflat = N[0].reshape(-1)
    egrain = _NWORKERS * _CHUNK
    etot = ((r * kappa + egrain - 1) // egrain) * egrain
    nflat = jnp.concatenate(
        [nflat, jnp.zeros((etot - r * kappa,), jnp.int32)])

    g = _sc_gather(table, nflat)

    qt = Q.T
    q2 = jnp.sum(Q * Q, axis=1)[None, :]
    wf = W.reshape(mq * din, dout)

    y = _tc_compute(g, x2, qt, q2, wf, r, dout)
    return y.reshape(k, r, dout)
